# Initial kernel scaffold; baseline (speedup 1.0000x reference)
import jax
import jax.numpy as jnp
from jax.experimental import pallas as pl


def kernel(x, table, W_out, b_out):
    raise NotImplementedError("write your pallas kernel here")


# speedup vs baseline: 8.3398x; 8.3398x over previous
import functools

import jax
import jax.numpy as jnp
from jax import lax
from jax.experimental import pallas as pl
from jax.experimental.pallas import tpu as pltpu
from jax.experimental.pallas import tpu_sc as plsc

B = 16384
F = 26
NROWS = B * F
EMB = 32
C_OUT = 128

NC = 2
NS = 16
NW = NC * NS
B_PER_W = NROWS // NW
CHUNK = 1664
N_CHUNKS = B_PER_W // CHUNK


@functools.partial(
    pl.kernel,
    mesh=plsc.VectorSubcoreMesh(core_axis_name="c", subcore_axis_name="s"),
    out_type=jax.ShapeDtypeStruct((NROWS, EMB), jnp.float32),
    scratch_types=[
        pltpu.VMEM((CHUNK,), jnp.int32),
        pltpu.VMEM((CHUNK, EMB), jnp.float32),
        pltpu.SemaphoreType.DMA,
    ],
    compiler_params=pltpu.CompilerParams(use_tc_tiling_on_sc=False),
)
def _sc_gather(idx_hbm, table_hbm, out_hbm, idx_v, rows_v, sem):
    wid = lax.axis_index("s") * NC + lax.axis_index("c")
    base = wid * B_PER_W
    for j in range(N_CHUNKS):
        off = base + j * CHUNK
        pltpu.sync_copy(idx_hbm.at[pl.ds(off, CHUNK)], idx_v)
        pltpu.async_copy(table_hbm.at[idx_v], rows_v, sem).wait()
        pltpu.sync_copy(rows_v, out_hbm.at[pl.ds(off, CHUNK)])


ROW_BLK = 2048
N_BLKS = NROWS // ROW_BLK


def _proj_body(emb_ref, wt_ref, b_ref, out_ref):
    out_ref[...] = (
        jnp.dot(emb_ref[...], wt_ref[...], preferred_element_type=jnp.float32)
        + b_ref[...]
    )


def _project(emb, w_t, b_row):
    return pl.pallas_call(
        _proj_body,
        grid=(N_BLKS,),
        in_specs=[
            pl.BlockSpec((ROW_BLK, EMB), lambda i: (i, 0)),
            pl.BlockSpec((EMB, C_OUT), lambda i: (0, 0)),
            pl.BlockSpec((1, C_OUT), lambda i: (0, 0)),
        ],
        out_specs=pl.BlockSpec((ROW_BLK, C_OUT), lambda i: (i, 0)),
        out_shape=jax.ShapeDtypeStruct((NROWS, C_OUT), jnp.float32),
    )(emb, w_t, b_row)


@jax.jit
def kernel(x, table, W_out, b_out):
    idx = x.reshape(-1).astype(jnp.int32)
    emb = _sc_gather(idx, table)
    out = _project(emb, W_out.T, b_out.reshape(1, C_OUT))
    return out.reshape(B, F, C_OUT)


# speedup vs baseline: 27.0000x; 3.2375x over previous
import functools

import jax
import jax.numpy as jnp
from jax import lax
from jax.experimental import pallas as pl
from jax.experimental.pallas import tpu as pltpu
from jax.experimental.pallas import tpu_sc as plsc

B = 16384
F = 26
NROWS = B * F
EMB = 32
C_OUT = 128
N_TOK = 1000000

NC = 2
NS = 16
NW = NC * NS
B_PER_W = NROWS // NW
CHUNK = 416
N_CHUNKS = B_PER_W // CHUNK


@functools.partial(
    pl.kernel,
    mesh=plsc.VectorSubcoreMesh(core_axis_name="c", subcore_axis_name="s"),
    out_type=jax.ShapeDtypeStruct((NROWS, C_OUT), jnp.float32),
    scratch_types=[
        pltpu.VMEM((CHUNK,), jnp.int32),
        pltpu.VMEM((CHUNK,), jnp.int32),
        pltpu.VMEM((CHUNK, C_OUT), jnp.float32),
        pltpu.VMEM((CHUNK, C_OUT), jnp.float32),
        pltpu.SemaphoreType.DMA,
        pltpu.SemaphoreType.DMA,
    ],
)
def _sc_gather(idx_hbm, p_hbm, out_hbm, idx_v0, idx_v1, rows_v0, rows_v1,
               gsem, ssem):
    wid = lax.axis_index("s") * NC + lax.axis_index("c")
    base = wid * B_PER_W
    idx_bufs = (idx_v0, idx_v1)
    row_bufs = (rows_v0, rows_v1)

    def start_gather(j):
        s = j % 2
        pltpu.sync_copy(idx_hbm.at[pl.ds(base + j * CHUNK, CHUNK)], idx_bufs[s])
        return pltpu.async_copy(p_hbm.at[idx_bufs[s]], row_bufs[s], gsem)

    gathers = {0: start_gather(0)}
    scatters = {}
    for j in range(N_CHUNKS):
        s = j % 2
        if j + 1 < N_CHUNKS:
            if j - 1 >= 0:
                scatters.pop(j - 1).wait()
            gathers[j + 1] = start_gather(j + 1)
        gathers.pop(j).wait()
        scatters[j] = pltpu.async_copy(
            row_bufs[s], out_hbm.at[pl.ds(base + j * CHUNK, CHUNK)], ssem
        )
    scatters.pop(N_CHUNKS - 1).wait()


TBLK = 8192
N_BLKS = -(-N_TOK // TBLK)


def _proj_body(t_ref, w_ref, b_ref, p_ref):
    p_ref[...] = (
        lax.dot_general(
            t_ref[...], w_ref[...],
            (((0,), (0,)), ((), ())),
            preferred_element_type=jnp.float32,
        )
        + b_ref[...]
    )


def _project_table(table_t, w_t, b_row):
    return pl.pallas_call(
        _proj_body,
        grid=(N_BLKS,),
        in_specs=[
            pl.BlockSpec((EMB, TBLK), lambda i: (0, i)),
            pl.BlockSpec((EMB, C_OUT), lambda i: (0, 0)),
            pl.BlockSpec((1, C_OUT), lambda i: (0, 0)),
        ],
        out_specs=pl.BlockSpec((TBLK, C_OUT), lambda i: (i, 0)),
        out_shape=jax.ShapeDtypeStruct((N_TOK, C_OUT), jnp.float32),
    )(table_t, w_t, b_row)


@jax.jit
def kernel(x, table, W_out, b_out):
    p = _project_table(table.T, W_out.T, b_out.reshape(1, C_OUT))
    idx = x.T.reshape(-1).astype(jnp.int32)
    gathered = _sc_gather(idx, p)
    return gathered.reshape(F, B, C_OUT).transpose(1, 0, 2)
